# Initial kernel scaffold; baseline (speedup 1.0000x reference)
import jax
import jax.numpy as jnp
from jax.experimental import pallas as pl


def kernel(x, edge_index, batch, params):
    raise NotImplementedError("write your pallas kernel here")


# speedup vs baseline: 2.6684x; 2.6684x over previous
import functools

import jax
import jax.numpy as jnp
from jax import lax
from jax.experimental import pallas as pl
from jax.experimental.pallas import tpu as pltpu
from jax.experimental.pallas import tpu_sc as plsc

_N = 10000
_E = 320000
_H = 128
_L = 4
_G = 128
_C = 10

_NC = 2
_NS = 16
_NW = _NC * _NS
_EPT = _E // _NW
_CH = 40
_NCH = _EPT // _CH
_BI = 8
_NB = _NCH // _BI
_RPT = 624
_TAIL0 = _NS * _RPT
_TAIL = _N - _TAIL0


@functools.cache
def _make_agg():
    mesh = plsc.VectorSubcoreMesh(core_axis_name="c", subcore_axis_name="s")

    @functools.partial(
        pl.kernel,
        mesh=mesh,
        out_type=jax.ShapeDtypeStruct((_NC, _N, _H), jnp.float32),
        scratch_types=[
            pltpu.VMEM((_CH,), jnp.int32),
            pltpu.VMEM((_CH,), jnp.int32),
            pltpu.VMEM((_CH, _H), jnp.float32),
            pltpu.VMEM_SHARED((_N, _H), jnp.float32),
        ],
    )
    def agg(h_hbm, src_hbm, dst_hbm, zeros_hbm, out_hbm, sidx, didx, rows, acc):
        c = lax.axis_index("c")
        s = lax.axis_index("s")
        wid = c * _NS + s
        r0 = s * _RPT

        @pl.when(c == 0)
        def _():
            pltpu.sync_copy(h_hbm.at[pl.ds(r0, _RPT)], acc.at[pl.ds(r0, _RPT)])

            @pl.when(s == 0)
            def _():
                pltpu.sync_copy(
                    h_hbm.at[pl.ds(_TAIL0, _TAIL)], acc.at[pl.ds(_TAIL0, _TAIL)]
                )

        @pl.when(c != 0)
        def _():
            pltpu.sync_copy(zeros_hbm.at[pl.ds(r0, _RPT)], acc.at[pl.ds(r0, _RPT)])

            @pl.when(s == 0)
            def _():
                pltpu.sync_copy(
                    zeros_hbm.at[pl.ds(_TAIL0, _TAIL)], acc.at[pl.ds(_TAIL0, _TAIL)]
                )

        plsc.subcore_barrier()

        @pl.loop(0, _NCH)
        def _(j):
            pltpu.sync_copy(src_hbm.at[wid, j], sidx)
            pltpu.sync_copy(dst_hbm.at[wid, j], didx)
            pltpu.sync_copy(h_hbm.at[sidx], rows)
            pltpu.sync_copy(rows, acc.at[didx], add=True)

        plsc.subcore_barrier()
        pltpu.sync_copy(acc.at[pl.ds(r0, _RPT)], out_hbm.at[c, pl.ds(r0, _RPT)])

        @pl.when(s == 0)
        def _():
            pltpu.sync_copy(
                acc.at[pl.ds(_TAIL0, _TAIL)], out_hbm.at[c, pl.ds(_TAIL0, _TAIL)]
            )

    return agg


def _layer_body(a0, a1, batch, w1, b1, g1, be1, w2, b2, g2, be2, z_out, pooled):
    zin = a0[...] + a1[...]
    z1 = jnp.dot(zin, w1[...], preferred_element_type=jnp.float32, precision=lax.Precision.HIGHEST) + b1[...]
    m1 = jnp.mean(z1, axis=0, keepdims=True)
    v1 = jnp.mean((z1 - m1) ** 2, axis=0, keepdims=True)
    z1 = jnp.maximum((z1 - m1) * lax.rsqrt(v1 + 1e-5) * g1[...] + be1[...], 0.0)
    z2 = jnp.dot(z1, w2[...], preferred_element_type=jnp.float32, precision=lax.Precision.HIGHEST) + b2[...]
    m2 = jnp.mean(z2, axis=0, keepdims=True)
    v2 = jnp.mean((z2 - m2) ** 2, axis=0, keepdims=True)
    z = jnp.maximum((z2 - m2) * lax.rsqrt(v2 + 1e-5) * g2[...] + be2[...], 0.0)
    z_out[...] = z
    onehot = (lax.broadcasted_iota(jnp.int32, (_G, _N), 0) == batch[...]).astype(
        jnp.float32
    )
    pooled[...] = jnp.dot(onehot, z, preferred_element_type=jnp.float32, precision=lax.Precision.HIGHEST)


_layer_tc = pl.pallas_call(
    _layer_body,
    out_shape=(
        jax.ShapeDtypeStruct((_N, _H), jnp.float32),
        jax.ShapeDtypeStruct((_G, _H), jnp.float32),
    ),
)


def _head_body(pc, fw1, fb1, fw2, fb2, out):
    hdn = jnp.maximum(
        jnp.dot(pc[...], fw1[...], preferred_element_type=jnp.float32, precision=lax.Precision.HIGHEST) + fb1[...], 0.0
    )
    out[...] = jnp.dot(hdn, fw2[...], preferred_element_type=jnp.float32, precision=lax.Precision.HIGHEST) + fb2[...]


_head_tc = pl.pallas_call(
    _head_body,
    out_shape=jax.ShapeDtypeStruct((_G, _C), jnp.float32),
)


def kernel(x, edge_index, batch, params):
    src = edge_index[0].reshape(_NW, _NCH, _CH)
    dst = edge_index[1].reshape(_NW, _NCH, _CH)
    zeros = jnp.zeros((_N, _H), jnp.float32)
    batch2d = batch.reshape(1, _N)

    agg = _make_agg()
    h = x
    pooled = []
    for i in range(_L):
        acc = agg(h, src, dst, zeros)
        h, p = _layer_tc(
            acc[0],
            acc[1],
            batch2d,
            params[f"w1_{i}"],
            params[f"b1_{i}"].reshape(1, _H),
            params[f"g1_{i}"].reshape(1, _H),
            params[f"be1_{i}"].reshape(1, _H),
            params[f"w2_{i}"],
            params[f"b2_{i}"].reshape(1, _H),
            params[f"g2_{i}"].reshape(1, _H),
            params[f"be2_{i}"].reshape(1, _H),
        )
        pooled.append(p)

    pc = jnp.concatenate(pooled, axis=-1)
    return _head_tc(
        pc,
        params["fc1_w"],
        params["fc1_b"].reshape(1, _H),
        params["fc2_w"],
        params["fc2_b"].reshape(1, _C),
    )


# speedup vs baseline: 5.1366x; 1.9250x over previous
import functools

import jax
import jax.numpy as jnp
from jax import lax
from jax.experimental import pallas as pl
from jax.experimental.pallas import tpu as pltpu
from jax.experimental.pallas import tpu_sc as plsc

_N = 10000
_E = 320000
_H = 128
_L = 4
_G = 128
_C = 10

_NC = 2
_NS = 16
_NW = _NC * _NS
_EPT = _E // _NW
_CH = 125
_NCH = _EPT // _CH
_BI = 8
_NB = _NCH // _BI
_RPT = 624
_TAIL0 = _NS * _RPT
_TAIL = _N - _TAIL0


@functools.cache
def _make_agg():
    mesh = plsc.VectorSubcoreMesh(core_axis_name="c", subcore_axis_name="s")

    @functools.partial(
        pl.kernel,
        mesh=mesh,
        out_type=jax.ShapeDtypeStruct((_NC, _N, _H), jnp.float32),
        scratch_types=[
            pltpu.VMEM((_CH,), jnp.int32),
            pltpu.VMEM((_CH,), jnp.int32),
            pltpu.VMEM((_CH, _H), jnp.float32),
            pltpu.VMEM_SHARED((_N, _H), jnp.float32),
        ],
    )
    def agg(h_hbm, src_hbm, dst_hbm, zeros_hbm, out_hbm, sidx, didx, rows, acc):
        c = lax.axis_index("c")
        s = lax.axis_index("s")
        wid = c * _NS + s
        r0 = s * _RPT

        @pl.when(c == 0)
        def _():
            pltpu.sync_copy(h_hbm.at[pl.ds(r0, _RPT)], acc.at[pl.ds(r0, _RPT)])

            @pl.when(s == 0)
            def _():
                pltpu.sync_copy(
                    h_hbm.at[pl.ds(_TAIL0, _TAIL)], acc.at[pl.ds(_TAIL0, _TAIL)]
                )

        @pl.when(c != 0)
        def _():
            pltpu.sync_copy(zeros_hbm.at[pl.ds(r0, _RPT)], acc.at[pl.ds(r0, _RPT)])

            @pl.when(s == 0)
            def _():
                pltpu.sync_copy(
                    zeros_hbm.at[pl.ds(_TAIL0, _TAIL)], acc.at[pl.ds(_TAIL0, _TAIL)]
                )

        plsc.subcore_barrier()

        @pl.loop(0, _NCH)
        def _(j):
            pltpu.sync_copy(src_hbm.at[wid, j], sidx)
            pltpu.sync_copy(dst_hbm.at[wid, j], didx)
            pltpu.sync_copy(h_hbm.at[sidx], rows)
            pltpu.sync_copy(rows, acc.at[didx], add=True)

        plsc.subcore_barrier()
        pltpu.sync_copy(acc.at[pl.ds(r0, _RPT)], out_hbm.at[c, pl.ds(r0, _RPT)])

        @pl.when(s == 0)
        def _():
            pltpu.sync_copy(
                acc.at[pl.ds(_TAIL0, _TAIL)], out_hbm.at[c, pl.ds(_TAIL0, _TAIL)]
            )

    return agg


def _layer_body(a0, a1, batch, w1, b1, g1, be1, w2, b2, g2, be2, z_out, pooled):
    zin = a0[...] + a1[...]
    z1 = jnp.dot(zin, w1[...], preferred_element_type=jnp.float32, precision=lax.Precision.HIGHEST) + b1[...]
    m1 = jnp.mean(z1, axis=0, keepdims=True)
    v1 = jnp.mean((z1 - m1) ** 2, axis=0, keepdims=True)
    z1 = jnp.maximum((z1 - m1) * lax.rsqrt(v1 + 1e-5) * g1[...] + be1[...], 0.0)
    z2 = jnp.dot(z1, w2[...], preferred_element_type=jnp.float32, precision=lax.Precision.HIGHEST) + b2[...]
    m2 = jnp.mean(z2, axis=0, keepdims=True)
    v2 = jnp.mean((z2 - m2) ** 2, axis=0, keepdims=True)
    z = jnp.maximum((z2 - m2) * lax.rsqrt(v2 + 1e-5) * g2[...] + be2[...], 0.0)
    z_out[...] = z
    onehot = (lax.broadcasted_iota(jnp.int32, (_G, _N), 0) == batch[...]).astype(
        jnp.float32
    )
    pooled[...] = jnp.dot(onehot, z, preferred_element_type=jnp.float32, precision=lax.Precision.HIGHEST)


_layer_tc = pl.pallas_call(
    _layer_body,
    out_shape=(
        jax.ShapeDtypeStruct((_N, _H), jnp.float32),
        jax.ShapeDtypeStruct((_G, _H), jnp.float32),
    ),
)


def _head_body(pc, fw1, fb1, fw2, fb2, out):
    hdn = jnp.maximum(
        jnp.dot(pc[...], fw1[...], preferred_element_type=jnp.float32, precision=lax.Precision.HIGHEST) + fb1[...], 0.0
    )
    out[...] = jnp.dot(hdn, fw2[...], preferred_element_type=jnp.float32, precision=lax.Precision.HIGHEST) + fb2[...]


_head_tc = pl.pallas_call(
    _head_body,
    out_shape=jax.ShapeDtypeStruct((_G, _C), jnp.float32),
)


def kernel(x, edge_index, batch, params):
    src = edge_index[0].reshape(_NW, _NCH, _CH)
    dst = edge_index[1].reshape(_NW, _NCH, _CH)
    zeros = jnp.zeros((_N, _H), jnp.float32)
    batch2d = batch.reshape(1, _N)

    agg = _make_agg()
    h = x
    pooled = []
    for i in range(_L):
        acc = agg(h, src, dst, zeros)
        h, p = _layer_tc(
            acc[0],
            acc[1],
            batch2d,
            params[f"w1_{i}"],
            params[f"b1_{i}"].reshape(1, _H),
            params[f"g1_{i}"].reshape(1, _H),
            params[f"be1_{i}"].reshape(1, _H),
            params[f"w2_{i}"],
            params[f"b2_{i}"].reshape(1, _H),
            params[f"g2_{i}"].reshape(1, _H),
            params[f"be2_{i}"].reshape(1, _H),
        )
        pooled.append(p)

    pc = jnp.concatenate(pooled, axis=-1)
    return _head_tc(
        pc,
        params["fc1_w"],
        params["fc1_b"].reshape(1, _H),
        params["fc2_w"],
        params["fc2_b"].reshape(1, _C),
    )


# speedup vs baseline: 8.5745x; 1.6693x over previous
import functools

import jax
import jax.numpy as jnp
from jax import lax
from jax.experimental import pallas as pl
from jax.experimental.pallas import tpu as pltpu
from jax.experimental.pallas import tpu_sc as plsc

_N = 10000
_E = 320000
_H = 128
_L = 4
_G = 128
_C = 10

_NC = 2
_NS = 16
_NW = _NC * _NS
_EPT = _E // _NW
_CH = 125
_NCH = _EPT // _CH
_NSUP = _NCH // 4
_RPT = 624
_TAIL0 = _NS * _RPT
_TAIL = _N - _TAIL0


@functools.cache
def _make_agg():
    mesh = plsc.VectorSubcoreMesh(core_axis_name="c", subcore_axis_name="s")

    @functools.partial(
        pl.kernel,
        mesh=mesh,
        out_type=jax.ShapeDtypeStruct((_NC, _N, _H), jnp.float32),
        scratch_types=(
            [pltpu.VMEM((_CH,), jnp.int32) for _ in range(8)]
            + [pltpu.VMEM((_CH, _H), jnp.float32) for _ in range(2)]
            + [pltpu.VMEM_SHARED((_N, _H), jnp.float32)]
            + [pltpu.SemaphoreType.DMA for _ in range(8)]
        ),
    )
    def agg(h_hbm, src_hbm, dst_hbm, zeros_hbm, out_hbm, *scr):
        sidx = scr[0:4]
        didx = scr[4:8]
        rows = scr[8:10]
        acc = scr[10]
        si = scr[11:15]
        gsem = scr[15:17]
        ssem = scr[17:19]
        c = lax.axis_index("c")
        s = lax.axis_index("s")
        wid = c * _NS + s
        r0 = s * _RPT

        @pl.when(c == 0)
        def _():
            pltpu.sync_copy(h_hbm.at[pl.ds(r0, _RPT)], acc.at[pl.ds(r0, _RPT)])

            @pl.when(s == 0)
            def _():
                pltpu.sync_copy(
                    h_hbm.at[pl.ds(_TAIL0, _TAIL)], acc.at[pl.ds(_TAIL0, _TAIL)]
                )

        @pl.when(c != 0)
        def _():
            pltpu.sync_copy(zeros_hbm.at[pl.ds(r0, _RPT)], acc.at[pl.ds(r0, _RPT)])

            @pl.when(s == 0)
            def _():
                pltpu.sync_copy(
                    zeros_hbm.at[pl.ds(_TAIL0, _TAIL)], acc.at[pl.ds(_TAIL0, _TAIL)]
                )

        plsc.subcore_barrier()

        def is_(j, q):
            pltpu.async_copy(src_hbm.at[wid, j], sidx[q], si[q])
            pltpu.async_copy(dst_hbm.at[wid, j], didx[q], si[q])

        def iw(q):
            pltpu.make_async_copy(src_hbm.at[wid, 0], sidx[q], si[q]).wait()
            pltpu.make_async_copy(dst_hbm.at[wid, 0], didx[q], si[q]).wait()

        def gs(q, r):
            pltpu.async_copy(h_hbm.at[sidx[q]], rows[r], gsem[r])

        def gw(q, r):
            pltpu.make_async_copy(h_hbm.at[sidx[q]], rows[r], gsem[r]).wait()

        def ss(q, r):
            pltpu.async_copy(rows[r], acc.at[didx[q]], ssem[r], add=True)

        def sw(q, r):
            pltpu.make_async_copy(rows[r], acc.at[didx[q]], ssem[r]).wait()

        for q in range(4):
            is_(q, q)
        iw(0)
        gs(0, 0)

        @pl.loop(0, _NSUP)
        def _(m):
            base = m * 4
            not_last = m < _NSUP - 1
            gw(0, 0)
            ss(0, 0)
            iw(1)
            gs(1, 1)
            sw(0, 0)

            @pl.when(not_last)
            def _():
                is_(base + 4, 0)

            gw(1, 1)
            ss(1, 1)
            iw(2)
            gs(2, 0)
            sw(1, 1)

            @pl.when(not_last)
            def _():
                is_(base + 5, 1)

            gw(2, 0)
            ss(2, 0)
            iw(3)
            gs(3, 1)
            sw(2, 0)

            @pl.when(not_last)
            def _():
                is_(base + 6, 2)

            gw(3, 1)
            ss(3, 1)

            @pl.when(not_last)
            def _():
                iw(0)
                gs(0, 0)

            sw(3, 1)

            @pl.when(not_last)
            def _():
                is_(base + 7, 3)

        plsc.subcore_barrier()
        pltpu.sync_copy(acc.at[pl.ds(r0, _RPT)], out_hbm.at[c, pl.ds(r0, _RPT)])

        @pl.when(s == 0)
        def _():
            pltpu.sync_copy(
                acc.at[pl.ds(_TAIL0, _TAIL)], out_hbm.at[c, pl.ds(_TAIL0, _TAIL)]
            )

    return agg


def _layer_body(a0, a1, batch, w1, b1, g1, be1, w2, b2, g2, be2, z_out, pooled):
    zin = a0[...] + a1[...]
    z1 = jnp.dot(zin, w1[...], preferred_element_type=jnp.float32, precision=lax.Precision.HIGHEST) + b1[...]
    m1 = jnp.mean(z1, axis=0, keepdims=True)
    v1 = jnp.mean((z1 - m1) ** 2, axis=0, keepdims=True)
    z1 = jnp.maximum((z1 - m1) * lax.rsqrt(v1 + 1e-5) * g1[...] + be1[...], 0.0)
    z2 = jnp.dot(z1, w2[...], preferred_element_type=jnp.float32, precision=lax.Precision.HIGHEST) + b2[...]
    m2 = jnp.mean(z2, axis=0, keepdims=True)
    v2 = jnp.mean((z2 - m2) ** 2, axis=0, keepdims=True)
    z = jnp.maximum((z2 - m2) * lax.rsqrt(v2 + 1e-5) * g2[...] + be2[...], 0.0)
    z_out[...] = z
    onehot = (lax.broadcasted_iota(jnp.int32, (_G, _N), 0) == batch[...]).astype(
        jnp.float32
    )
    pooled[...] = jnp.dot(onehot, z, preferred_element_type=jnp.float32, precision=lax.Precision.HIGHEST)


_layer_tc = pl.pallas_call(
    _layer_body,
    out_shape=(
        jax.ShapeDtypeStruct((_N, _H), jnp.float32),
        jax.ShapeDtypeStruct((_G, _H), jnp.float32),
    ),
)


def _head_body(pc, fw1, fb1, fw2, fb2, out):
    hdn = jnp.maximum(
        jnp.dot(pc[...], fw1[...], preferred_element_type=jnp.float32, precision=lax.Precision.HIGHEST) + fb1[...], 0.0
    )
    out[...] = jnp.dot(hdn, fw2[...], preferred_element_type=jnp.float32, precision=lax.Precision.HIGHEST) + fb2[...]


_head_tc = pl.pallas_call(
    _head_body,
    out_shape=jax.ShapeDtypeStruct((_G, _C), jnp.float32),
)


def kernel(x, edge_index, batch, params):
    src = edge_index[0].reshape(_NW, _NCH, _CH)
    dst = edge_index[1].reshape(_NW, _NCH, _CH)
    zeros = jnp.zeros((_N, _H), jnp.float32)
    batch2d = batch.reshape(1, _N)

    agg = _make_agg()
    h = x
    pooled = []
    for i in range(_L):
        acc = agg(h, src, dst, zeros)
        h, p = _layer_tc(
            acc[0],
            acc[1],
            batch2d,
            params[f"w1_{i}"],
            params[f"b1_{i}"].reshape(1, _H),
            params[f"g1_{i}"].reshape(1, _H),
            params[f"be1_{i}"].reshape(1, _H),
            params[f"w2_{i}"],
            params[f"b2_{i}"].reshape(1, _H),
            params[f"g2_{i}"].reshape(1, _H),
            params[f"be2_{i}"].reshape(1, _H),
        )
        pooled.append(p)

    pc = jnp.concatenate(pooled, axis=-1)
    return _head_tc(
        pc,
        params["fc1_w"],
        params["fc1_b"].reshape(1, _H),
        params["fc2_w"],
        params["fc2_b"].reshape(1, _C),
    )


# speedup vs baseline: 8.6054x; 1.0036x over previous
import functools

import jax
import jax.numpy as jnp
from jax import lax
from jax.experimental import pallas as pl
from jax.experimental.pallas import tpu as pltpu
from jax.experimental.pallas import tpu_sc as plsc

_N = 10000
_E = 320000
_H = 128
_L = 4
_G = 128
_C = 10

_NC = 2
_NS = 16
_NW = _NC * _NS
_EPT = _E // _NW
_CH = 125
_NCH = _EPT // _CH
_NSUP = _NCH // 4
_RPT = 624
_TAIL0 = _NS * _RPT
_TAIL = _N - _TAIL0


@functools.cache
def _make_agg():
    mesh = plsc.VectorSubcoreMesh(core_axis_name="c", subcore_axis_name="s")

    @functools.partial(
        pl.kernel,
        mesh=mesh,
        out_type=jax.ShapeDtypeStruct((_NC, _N, _H), jnp.float32),
        scratch_types=(
            [pltpu.VMEM((_CH,), jnp.int32) for _ in range(8)]
            + [pltpu.VMEM((_CH, _H), jnp.float32) for _ in range(2)]
            + [pltpu.VMEM_SHARED((_N, _H), jnp.float32)]
            + [pltpu.SemaphoreType.DMA for _ in range(8)]
        ),
    )
    def agg(h_hbm, src_hbm, dst_hbm, zeros_hbm, out_hbm, *scr):
        sidx = scr[0:4]
        didx = scr[4:8]
        rows = scr[8:10]
        acc = scr[10]
        si = scr[11:15]
        gsem = scr[15:17]
        ssem = scr[17:19]
        c = lax.axis_index("c")
        s = lax.axis_index("s")
        wid = c * _NS + s
        r0 = s * _RPT

        @pl.when(c == 0)
        def _():
            pltpu.sync_copy(h_hbm.at[pl.ds(r0, _RPT)], acc.at[pl.ds(r0, _RPT)])

            @pl.when(s == 0)
            def _():
                pltpu.sync_copy(
                    h_hbm.at[pl.ds(_TAIL0, _TAIL)], acc.at[pl.ds(_TAIL0, _TAIL)]
                )

        @pl.when(c != 0)
        def _():
            pltpu.sync_copy(zeros_hbm.at[pl.ds(r0, _RPT)], acc.at[pl.ds(r0, _RPT)])

            @pl.when(s == 0)
            def _():
                pltpu.sync_copy(
                    zeros_hbm.at[pl.ds(_TAIL0, _TAIL)], acc.at[pl.ds(_TAIL0, _TAIL)]
                )

        plsc.subcore_barrier()

        def is_(j, q):
            pltpu.async_copy(src_hbm.at[wid, j], sidx[q], si[q])
            pltpu.async_copy(dst_hbm.at[wid, j], didx[q], si[q])

        def iw(q):
            pltpu.make_async_copy(src_hbm.at[wid, 0], sidx[q], si[q]).wait()
            pltpu.make_async_copy(dst_hbm.at[wid, 0], didx[q], si[q]).wait()

        def gs(q, r):
            pltpu.async_copy(h_hbm.at[sidx[q]], rows[r], gsem[r])

        def gw(q, r):
            pltpu.make_async_copy(h_hbm.at[sidx[q]], rows[r], gsem[r]).wait()

        def ss(q, r):
            pltpu.async_copy(rows[r], acc.at[didx[q]], ssem[r], add=True)

        def sw(q, r):
            pltpu.make_async_copy(rows[r], acc.at[didx[q]], ssem[r]).wait()

        for q in range(4):
            is_(q, q)
        iw(0)
        gs(0, 0)

        @pl.loop(0, _NSUP)
        def _(m):
            base = m * 4
            not_last = m < _NSUP - 1
            gw(0, 0)
            ss(0, 0)
            iw(1)
            gs(1, 1)
            sw(0, 0)

            @pl.when(not_last)
            def _():
                is_(base + 4, 0)

            gw(1, 1)
            ss(1, 1)
            iw(2)
            gs(2, 0)
            sw(1, 1)

            @pl.when(not_last)
            def _():
                is_(base + 5, 1)

            gw(2, 0)
            ss(2, 0)
            iw(3)
            gs(3, 1)
            sw(2, 0)

            @pl.when(not_last)
            def _():
                is_(base + 6, 2)

            gw(3, 1)
            ss(3, 1)

            @pl.when(not_last)
            def _():
                iw(0)
                gs(0, 0)

            sw(3, 1)

            @pl.when(not_last)
            def _():
                is_(base + 7, 3)

        plsc.subcore_barrier()
        pltpu.sync_copy(acc.at[pl.ds(r0, _RPT)], out_hbm.at[c, pl.ds(r0, _RPT)])

        @pl.when(s == 0)
        def _():
            pltpu.sync_copy(
                acc.at[pl.ds(_TAIL0, _TAIL)], out_hbm.at[c, pl.ds(_TAIL0, _TAIL)]
            )

    return agg


def _layer_body(a0, a1, batch, w1, b1, g1, be1, w2, b2, g2, be2, z_out, pooled):
    zin = a0[...] + a1[...]
    z1 = jnp.dot(zin, w1[...], preferred_element_type=jnp.float32, precision=lax.Precision.HIGHEST) + b1[...]
    m1 = jnp.mean(z1, axis=0, keepdims=True)
    v1 = jnp.mean((z1 - m1) ** 2, axis=0, keepdims=True)
    z1 = jnp.maximum((z1 - m1) * lax.rsqrt(v1 + 1e-5) * g1[...] + be1[...], 0.0)
    z2 = jnp.dot(z1, w2[...], preferred_element_type=jnp.float32, precision=lax.Precision.HIGHEST) + b2[...]
    m2 = jnp.mean(z2, axis=0, keepdims=True)
    v2 = jnp.mean((z2 - m2) ** 2, axis=0, keepdims=True)
    z = jnp.maximum((z2 - m2) * lax.rsqrt(v2 + 1e-5) * g2[...] + be2[...], 0.0)
    z_out[...] = z
    onehot = (lax.broadcasted_iota(jnp.int32, (_G, _N), 0) == batch[...]).astype(
        jnp.float32
    )
    pooled[...] = jnp.dot(onehot, z, preferred_element_type=jnp.float32, precision=lax.Precision.HIGHEST)


_layer_tc = pl.pallas_call(
    _layer_body,
    out_shape=(
        jax.ShapeDtypeStruct((_N, _H), jnp.float32),
        jax.ShapeDtypeStruct((_G, _H), jnp.float32),
    ),
)


def _last_body(a0, a1, batch, w1, b1, g1, be1, w2, b2, g2, be2,
               p0, p1, p2, fw10, fw11, fw12, fw13, fb1, fw2, fb2, out):
    zin = a0[...] + a1[...]
    z1 = jnp.dot(zin, w1[...], preferred_element_type=jnp.float32, precision=lax.Precision.HIGHEST) + b1[...]
    m1 = jnp.mean(z1, axis=0, keepdims=True)
    v1 = jnp.mean((z1 - m1) ** 2, axis=0, keepdims=True)
    z1 = jnp.maximum((z1 - m1) * lax.rsqrt(v1 + 1e-5) * g1[...] + be1[...], 0.0)
    z2 = jnp.dot(z1, w2[...], preferred_element_type=jnp.float32, precision=lax.Precision.HIGHEST) + b2[...]
    m2 = jnp.mean(z2, axis=0, keepdims=True)
    v2 = jnp.mean((z2 - m2) ** 2, axis=0, keepdims=True)
    z = jnp.maximum((z2 - m2) * lax.rsqrt(v2 + 1e-5) * g2[...] + be2[...], 0.0)
    onehot = (lax.broadcasted_iota(jnp.int32, (_G, _N), 0) == batch[...]).astype(
        jnp.float32
    )
    p3 = jnp.dot(onehot, z, preferred_element_type=jnp.float32, precision=lax.Precision.HIGHEST)
    hdn = (
        jnp.dot(p0[...], fw10[...], preferred_element_type=jnp.float32, precision=lax.Precision.HIGHEST)
        + jnp.dot(p1[...], fw11[...], preferred_element_type=jnp.float32, precision=lax.Precision.HIGHEST)
        + jnp.dot(p2[...], fw12[...], preferred_element_type=jnp.float32, precision=lax.Precision.HIGHEST)
        + jnp.dot(p3, fw13[...], preferred_element_type=jnp.float32, precision=lax.Precision.HIGHEST)
        + fb1[...]
    )
    hdn = jnp.maximum(hdn, 0.0)
    out[...] = jnp.dot(hdn, fw2[...], preferred_element_type=jnp.float32, precision=lax.Precision.HIGHEST) + fb2[...]


_last_tc = pl.pallas_call(
    _last_body,
    out_shape=jax.ShapeDtypeStruct((_G, _C), jnp.float32),
)


def kernel(x, edge_index, batch, params):
    src = edge_index[0].reshape(_NW, _NCH, _CH)
    dst = edge_index[1].reshape(_NW, _NCH, _CH)
    zeros = jnp.zeros((_N, _H), jnp.float32)
    batch2d = batch.reshape(1, _N)

    agg = _make_agg()
    h = x
    pooled = []
    for i in range(_L - 1):
        acc = agg(h, src, dst, zeros)
        h, p = _layer_tc(
            acc[0],
            acc[1],
            batch2d,
            params[f"w1_{i}"],
            params[f"b1_{i}"].reshape(1, _H),
            params[f"g1_{i}"].reshape(1, _H),
            params[f"be1_{i}"].reshape(1, _H),
            params[f"w2_{i}"],
            params[f"b2_{i}"].reshape(1, _H),
            params[f"g2_{i}"].reshape(1, _H),
            params[f"be2_{i}"].reshape(1, _H),
        )
        pooled.append(p)

    i = _L - 1
    acc = agg(h, src, dst, zeros)
    fw1 = params["fc1_w"]
    return _last_tc(
        acc[0],
        acc[1],
        batch2d,
        params[f"w1_{i}"],
        params[f"b1_{i}"].reshape(1, _H),
        params[f"g1_{i}"].reshape(1, _H),
        params[f"be1_{i}"].reshape(1, _H),
        params[f"w2_{i}"],
        params[f"b2_{i}"].reshape(1, _H),
        params[f"g2_{i}"].reshape(1, _H),
        params[f"be2_{i}"].reshape(1, _H),
        pooled[0],
        pooled[1],
        pooled[2],
        fw1[0 * _H : 1 * _H],
        fw1[1 * _H : 2 * _H],
        fw1[2 * _H : 3 * _H],
        fw1[3 * _H : 4 * _H],
        params["fc1_b"].reshape(1, _H),
        params["fc2_w"],
        params["fc2_b"].reshape(1, _C),
    )


# speedup vs baseline: 8.8175x; 1.0246x over previous
import functools

import jax
import jax.numpy as jnp
from jax import lax
from jax.experimental import pallas as pl
from jax.experimental.pallas import tpu as pltpu
from jax.experimental.pallas import tpu_sc as plsc

_N = 10000
_E = 320000
_H = 128
_L = 4
_G = 128
_C = 10

_NC = 2
_NS = 16
_NW = _NC * _NS
_EPT = _E // _NW
_CH = 125
_NCH = _EPT // _CH
_NSUP = _NCH // 4
_RPT = 624
_TAIL0 = _NS * _RPT
_TAIL = _N - _TAIL0


@functools.cache
def _make_agg():
    mesh = plsc.VectorSubcoreMesh(core_axis_name="c", subcore_axis_name="s")

    @functools.partial(
        pl.kernel,
        mesh=mesh,
        out_type=jax.ShapeDtypeStruct((_NC, _N, _H), jnp.float32),
        scratch_types=(
            [pltpu.VMEM((_CH,), jnp.int32) for _ in range(8)]
            + [pltpu.VMEM((_CH, _H), jnp.float32) for _ in range(2)]
            + [pltpu.VMEM_SHARED((_N, _H), jnp.float32)]
            + [pltpu.SemaphoreType.DMA for _ in range(8)]
        ),
    )
    def agg(h_hbm, src_hbm, dst_hbm, zeros_hbm, out_hbm, *scr):
        sidx = scr[0:4]
        didx = scr[4:8]
        rows = scr[8:10]
        acc = scr[10]
        si = scr[11:15]
        gsem = scr[15:17]
        ssem = scr[17:19]
        c = lax.axis_index("c")
        s = lax.axis_index("s")
        wid = c * _NS + s
        r0 = s * _RPT

        def is0_(j, q):
            pltpu.async_copy(src_hbm.at[wid, j], scr[q], scr[11 + q])
            pltpu.async_copy(dst_hbm.at[wid, j], scr[4 + q], scr[11 + q])

        for q in range(4):
            is0_(q, q)
        pltpu.make_async_copy(src_hbm.at[wid, 0], scr[0], scr[11]).wait()
        pltpu.make_async_copy(dst_hbm.at[wid, 0], scr[4], scr[11]).wait()
        pltpu.async_copy(h_hbm.at[scr[0]], scr[8], scr[15])

        @pl.when(c == 0)
        def _():
            pltpu.sync_copy(h_hbm.at[pl.ds(r0, _RPT)], acc.at[pl.ds(r0, _RPT)])

            @pl.when(s == 0)
            def _():
                pltpu.sync_copy(
                    h_hbm.at[pl.ds(_TAIL0, _TAIL)], acc.at[pl.ds(_TAIL0, _TAIL)]
                )

        @pl.when(c != 0)
        def _():
            pltpu.sync_copy(zeros_hbm.at[pl.ds(r0, _RPT)], acc.at[pl.ds(r0, _RPT)])

            @pl.when(s == 0)
            def _():
                pltpu.sync_copy(
                    zeros_hbm.at[pl.ds(_TAIL0, _TAIL)], acc.at[pl.ds(_TAIL0, _TAIL)]
                )

        plsc.subcore_barrier()

        def is_(j, q):
            pltpu.async_copy(src_hbm.at[wid, j], sidx[q], si[q])
            pltpu.async_copy(dst_hbm.at[wid, j], didx[q], si[q])

        def iw(q):
            pltpu.make_async_copy(src_hbm.at[wid, 0], sidx[q], si[q]).wait()
            pltpu.make_async_copy(dst_hbm.at[wid, 0], didx[q], si[q]).wait()

        def gs(q, r):
            pltpu.async_copy(h_hbm.at[sidx[q]], rows[r], gsem[r])

        def gw(q, r):
            pltpu.make_async_copy(h_hbm.at[sidx[q]], rows[r], gsem[r]).wait()

        def ss(q, r):
            pltpu.async_copy(rows[r], acc.at[didx[q]], ssem[r], add=True)

        def sw(q, r):
            pltpu.make_async_copy(rows[r], acc.at[didx[q]], ssem[r]).wait()

        @pl.loop(0, _NSUP)
        def _(m):
            base = m * 4
            not_last = m < _NSUP - 1
            gw(0, 0)
            ss(0, 0)
            iw(1)
            gs(1, 1)
            sw(0, 0)

            @pl.when(not_last)
            def _():
                is_(base + 4, 0)

            gw(1, 1)
            ss(1, 1)
            iw(2)
            gs(2, 0)
            sw(1, 1)

            @pl.when(not_last)
            def _():
                is_(base + 5, 1)

            gw(2, 0)
            ss(2, 0)
            iw(3)
            gs(3, 1)
            sw(2, 0)

            @pl.when(not_last)
            def _():
                is_(base + 6, 2)

            gw(3, 1)
            ss(3, 1)

            @pl.when(not_last)
            def _():
                iw(0)
                gs(0, 0)

            sw(3, 1)

            @pl.when(not_last)
            def _():
                is_(base + 7, 3)

        plsc.subcore_barrier()
        pltpu.sync_copy(acc.at[pl.ds(r0, _RPT)], out_hbm.at[c, pl.ds(r0, _RPT)])

        @pl.when(s == 0)
        def _():
            pltpu.sync_copy(
                acc.at[pl.ds(_TAIL0, _TAIL)], out_hbm.at[c, pl.ds(_TAIL0, _TAIL)]
            )

    return agg


def _layer_body(a0, a1, batch, w1, b1, g1, be1, w2, b2, g2, be2, z_out, pooled):
    zin = a0[...] + a1[...]
    z1 = jnp.dot(zin, w1[...], preferred_element_type=jnp.float32, precision=lax.Precision.HIGHEST) + b1[...]
    m1 = jnp.mean(z1, axis=0, keepdims=True)
    v1 = jnp.mean((z1 - m1) ** 2, axis=0, keepdims=True)
    z1 = jnp.maximum((z1 - m1) * lax.rsqrt(v1 + 1e-5) * g1[...] + be1[...], 0.0)
    z2 = jnp.dot(z1, w2[...], preferred_element_type=jnp.float32, precision=lax.Precision.HIGHEST) + b2[...]
    m2 = jnp.mean(z2, axis=0, keepdims=True)
    v2 = jnp.mean((z2 - m2) ** 2, axis=0, keepdims=True)
    z = jnp.maximum((z2 - m2) * lax.rsqrt(v2 + 1e-5) * g2[...] + be2[...], 0.0)
    z_out[...] = z
    onehot = (lax.broadcasted_iota(jnp.int32, (_G, _N), 0) == batch[...]).astype(
        jnp.float32
    )
    pooled[...] = jnp.dot(onehot, z, preferred_element_type=jnp.float32)


_layer_tc = pl.pallas_call(
    _layer_body,
    out_shape=(
        jax.ShapeDtypeStruct((_N, _H), jnp.float32),
        jax.ShapeDtypeStruct((_G, _H), jnp.float32),
    ),
)


def _last_body(a0, a1, batch, w1, b1, g1, be1, w2, b2, g2, be2,
               p0, p1, p2, fw10, fw11, fw12, fw13, fb1, fw2, fb2, out):
    zin = a0[...] + a1[...]
    z1 = jnp.dot(zin, w1[...], preferred_element_type=jnp.float32, precision=lax.Precision.HIGHEST) + b1[...]
    m1 = jnp.mean(z1, axis=0, keepdims=True)
    v1 = jnp.mean((z1 - m1) ** 2, axis=0, keepdims=True)
    z1 = jnp.maximum((z1 - m1) * lax.rsqrt(v1 + 1e-5) * g1[...] + be1[...], 0.0)
    z2 = jnp.dot(z1, w2[...], preferred_element_type=jnp.float32, precision=lax.Precision.HIGHEST) + b2[...]
    m2 = jnp.mean(z2, axis=0, keepdims=True)
    v2 = jnp.mean((z2 - m2) ** 2, axis=0, keepdims=True)
    z = jnp.maximum((z2 - m2) * lax.rsqrt(v2 + 1e-5) * g2[...] + be2[...], 0.0)
    onehot = (lax.broadcasted_iota(jnp.int32, (_G, _N), 0) == batch[...]).astype(
        jnp.float32
    )
    p3 = jnp.dot(onehot, z, preferred_element_type=jnp.float32)
    hdn = (
        jnp.dot(p0[...], fw10[...], preferred_element_type=jnp.float32, precision=lax.Precision.HIGHEST)
        + jnp.dot(p1[...], fw11[...], preferred_element_type=jnp.float32, precision=lax.Precision.HIGHEST)
        + jnp.dot(p2[...], fw12[...], preferred_element_type=jnp.float32, precision=lax.Precision.HIGHEST)
        + jnp.dot(p3, fw13[...], preferred_element_type=jnp.float32, precision=lax.Precision.HIGHEST)
        + fb1[...]
    )
    hdn = jnp.maximum(hdn, 0.0)
    out[...] = jnp.dot(hdn, fw2[...], preferred_element_type=jnp.float32, precision=lax.Precision.HIGHEST) + fb2[...]


_last_tc = pl.pallas_call(
    _last_body,
    out_shape=jax.ShapeDtypeStruct((_G, _C), jnp.float32),
)


def kernel(x, edge_index, batch, params):
    src = edge_index[0].reshape(_NW, _NCH, _CH)
    dst = edge_index[1].reshape(_NW, _NCH, _CH)
    zeros = jnp.zeros((_N, _H), jnp.float32)
    batch2d = batch.reshape(1, _N)

    agg = _make_agg()
    h = x
    pooled = []
    for i in range(_L - 1):
        acc = agg(h, src, dst, zeros)
        h, p = _layer_tc(
            acc[0],
            acc[1],
            batch2d,
            params[f"w1_{i}"],
            params[f"b1_{i}"].reshape(1, _H),
            params[f"g1_{i}"].reshape(1, _H),
            params[f"be1_{i}"].reshape(1, _H),
            params[f"w2_{i}"],
            params[f"b2_{i}"].reshape(1, _H),
            params[f"g2_{i}"].reshape(1, _H),
            params[f"be2_{i}"].reshape(1, _H),
        )
        pooled.append(p)

    i = _L - 1
    acc = agg(h, src, dst, zeros)
    fw1 = params["fc1_w"]
    return _last_tc(
        acc[0],
        acc[1],
        batch2d,
        params[f"w1_{i}"],
        params[f"b1_{i}"].reshape(1, _H),
        params[f"g1_{i}"].reshape(1, _H),
        params[f"be1_{i}"].reshape(1, _H),
        params[f"w2_{i}"],
        params[f"b2_{i}"].reshape(1, _H),
        params[f"g2_{i}"].reshape(1, _H),
        params[f"be2_{i}"].reshape(1, _H),
        pooled[0],
        pooled[1],
        pooled[2],
        fw1[0 * _H : 1 * _H],
        fw1[1 * _H : 2 * _H],
        fw1[2 * _H : 3 * _H],
        fw1[3 * _H : 4 * _H],
        params["fc1_b"].reshape(1, _H),
        params["fc2_w"],
        params["fc2_b"].reshape(1, _C),
    )
